# baseline (speedup 1.0000x reference)
import dataclasses
import functools

import jax
import jax.numpy as jnp
from jax import lax
from jax.experimental import pallas as pl
from jax.experimental.pallas import tpu as pltpu
from jax.experimental.pallas import tpu_sc as plsc

NN = 10000
NP = 10240
D = 128
EE = 320000
NC = 2
NS = 16
NW = NC * NS
CH = 80
NCH = 125
EPW = NCH * CH
RPT = NP // NS

_mesh = plsc.VectorSubcoreMesh(core_axis_name="c", subcore_axis_name="s")

_cp = pltpu.CompilerParams()
if "needs_layout_passes" in pltpu.CompilerParams.__dataclass_fields__:
    _cp = dataclasses.replace(_cp, needs_layout_passes=False)


@functools.partial(
    pl.kernel,
    out_type=jax.ShapeDtypeStruct((NC, NP), jnp.float32),
    mesh=_mesh,
    compiler_params=_cp,
    scratch_types=[
        pltpu.VMEM((32, CH), jnp.int32),
        pltpu.VMEM((2 * CH,), jnp.float32),
        pltpu.VMEM((RPT,), jnp.float32),
        pltpu.VMEM_SHARED((NP,), jnp.float32),
        pltpu.SemaphoreType.DMA((2,)),
        pltpu.SemaphoreType.DMA((4,)),
    ],
)
def _deg_kernel(idx_hbm, out_hbm, ibuf, stage, zbuf, acc_s, ssem, isem):
    cid = lax.axis_index("c")
    sid = lax.axis_index("s")
    wid = sid * NC + cid

    zeros = jnp.zeros((16,), jnp.float32)

    @pl.loop(0, RPT // 16)
    def _(i):
        zbuf[pl.ds(pl.multiple_of(i * 16, 16), 16)] = zeros

    @pl.loop(0, 2 * CH // 16)
    def _(i):
        stage[pl.ds(pl.multiple_of(i * 16, 16), 16)] = zeros

    base = pl.multiple_of(sid * RPT, RPT)
    pltpu.sync_copy(zbuf, acc_s.at[pl.ds(base, RPT)])
    plsc.subcore_barrier()

    for t in range(4):
        pltpu.async_copy(idx_hbm.at[wid, t], ibuf.at[pl.ds(t * 8, 8)],
                         isem.at[t])
    for b in range(2):
        pltpu.make_async_copy(idx_hbm.at[wid, b],
                              ibuf.at[pl.ds(b * 8, 8)], isem.at[b]).wait()
        pltpu.async_copy(stage.at[pl.ds(b * CH, CH)],
                         acc_s.at[ibuf.at[b * 8 + 1]], ssem.at[b], add=True)

    @pl.loop(0, NCH)
    def _(j):
        b4 = j % 4
        b2 = j % 2
        ir = b4 * 8
        st = stage.at[pl.ds(b2 * CH, CH)]

        @pl.when(j >= 2)
        def _():
            pltpu.make_async_copy(idx_hbm.at[wid, 0],
                                  ibuf.at[pl.ds(b4 * 8, 8)], isem.at[b4]).wait()

        pltpu.make_async_copy(st, acc_s.at[ibuf.at[ir + 1]],
                              ssem.at[b2]).wait()

        @pl.loop(0, CH // 16)
        def _(k):
            sl = pl.ds(pl.multiple_of(k * 16, 16), 16)
            st[sl] = plsc.bitcast(ibuf[ir + 2, sl], jnp.float32)

        pltpu.async_copy(st, acc_s.at[ibuf.at[ir + 1]], ssem.at[b2], add=True)

        @pl.when(j + 4 < NCH)
        def _():
            pltpu.async_copy(idx_hbm.at[wid, j + 4],
                             ibuf.at[pl.ds(b4 * 8, 8)], isem.at[b4])

    for b in range(2):
        pltpu.make_async_copy(stage.at[pl.ds(b * CH, CH)],
                              acc_s.at[ibuf.at[b * 8 + 1]], ssem.at[b]).wait()

    plsc.subcore_barrier()

    pltpu.sync_copy(acc_s.at[pl.ds(base, RPT)], zbuf)
    pltpu.sync_copy(zbuf, out_hbm.at[cid, pl.ds(base, RPT)])


@functools.partial(
    pl.kernel,
    out_type=jax.ShapeDtypeStruct((NC, NP, D), jnp.float32),
    mesh=_mesh,
    compiler_params=_cp,
    scratch_types=[
        pltpu.VMEM((32, CH), jnp.int32),
        pltpu.VMEM((2 * CH, D), jnp.float32),
        pltpu.VMEM((2 * CH, D), jnp.float32),
        pltpu.VMEM_SHARED((NP, D), jnp.float32),
        pltpu.SemaphoreType.DMA((2,)),
        pltpu.SemaphoreType.DMA((2,)),
        pltpu.SemaphoreType.DMA((4,)),
    ],
)
def _agg_kernel(h_hbm, idx_hbm, out_hbm, ibuf, gbuf, sbuf, acc_s,
                gsem, ssem, isem):
    cid = lax.axis_index("c")
    sid = lax.axis_index("s")
    wid = sid * NC + cid

    zeros = jnp.zeros((16,), jnp.float32)

    @pl.loop(0, 2 * CH)
    def _(r):
        for f in range(D // 16):
            sbuf[r, pl.ds(f * 16, 16)] = zeros

    @pl.loop(0, RPT // CH)
    def _(t):
        base = pl.multiple_of(sid * RPT + t * CH, CH)
        pltpu.sync_copy(sbuf.at[pl.ds(0, CH)], acc_s.at[pl.ds(base, CH)])

    plsc.subcore_barrier()

    for t in range(4):
        pltpu.async_copy(idx_hbm.at[wid, t], ibuf.at[pl.ds(t * 8, 8)],
                         isem.at[t])
    for b in range(2):
        pltpu.make_async_copy(idx_hbm.at[wid, b],
                              ibuf.at[pl.ds(b * 8, 8)], isem.at[b]).wait()
        pltpu.async_copy(h_hbm.at[ibuf.at[b * 8]],
                         gbuf.at[pl.ds(b * CH, CH)], gsem.at[b])
        pltpu.async_copy(sbuf.at[pl.ds(b * CH, CH)],
                         acc_s.at[ibuf.at[b * 8 + 1]], ssem.at[b], add=True)

    @pl.loop(0, NCH)
    def _(j):
        b4 = j % 4
        b2 = j % 2
        g = gbuf.at[pl.ds(b2 * CH, CH)]
        s = sbuf.at[pl.ds(b2 * CH, CH)]
        ir = b4 * 8
        pltpu.make_async_copy(h_hbm.at[ibuf.at[ir]], g, gsem.at[b2]).wait()
        pltpu.make_async_copy(s, acc_s.at[ibuf.at[ir + 1]],
                              ssem.at[b2]).wait()

        @plsc.parallel_loop(0, CH // 16, unroll=5)
        def _(k):
            sl16 = pl.ds(pl.multiple_of(k * 16, 16), 16)
            wv = plsc.bitcast(ibuf[ir + 2, sl16], jnp.float32)
            for l in range(16):
                w = wv[l]
                e = k * 16 + l
                for f in range(D // 16):
                    sl = pl.ds(f * 16, 16)
                    s[e, sl] = g[e, sl] * w

        @pl.when(j + 2 < NCH)
        def _():
            bn = (b4 + 2) % 4
            pltpu.make_async_copy(idx_hbm.at[wid, 0],
                                  ibuf.at[pl.ds(bn * 8, 8)], isem.at[bn]).wait()
            pltpu.async_copy(h_hbm.at[ibuf.at[bn * 8]], g, gsem.at[b2])

        pltpu.async_copy(s, acc_s.at[ibuf.at[ir + 1]], ssem.at[b2], add=True)

        @pl.when(j + 4 < NCH)
        def _():
            pltpu.async_copy(idx_hbm.at[wid, j + 4],
                             ibuf.at[pl.ds(b4 * 8, 8)], isem.at[b4])

    for b in range(2):
        pltpu.make_async_copy(sbuf.at[pl.ds(b * CH, CH)],
                              acc_s.at[ibuf.at[b * 8 + 1]], ssem.at[b]).wait()

    plsc.subcore_barrier()

    @pl.loop(0, RPT // CH)
    def _(t):
        base = pl.multiple_of(sid * RPT + t * CH, CH)
        pltpu.sync_copy(acc_s.at[pl.ds(base, CH)], sbuf.at[pl.ds(0, CH)])
        pltpu.sync_copy(sbuf.at[pl.ds(0, CH)], out_hbm.at[cid, pl.ds(base, CH)])


def _mm_body(x_ref, w_ref, h_ref):
    h_ref[...] = jnp.dot(x_ref[...], w_ref[...],
                         preferred_element_type=jnp.float32)


def _scale_body(h_ref, degp_ref, o_ref):
    deg = 1.0 + jnp.sum(degp_ref[...], axis=0)
    o_ref[...] = h_ref[...] * lax.rsqrt(deg)[:, None]


def _midf_body(p_ref, h_ref, degp_ref, b_ref, w_ref, x_ref, hn_ref):
    deg = 1.0 + jnp.sum(degp_ref[...], axis=0)
    dis = lax.rsqrt(deg)[:, None]
    x2 = jnp.maximum(dis * (p_ref[0] + p_ref[1] + h_ref[...]) + b_ref[...], 0.0)
    x_ref[...] = x2
    hn = jnp.dot(x2, w_ref[...], preferred_element_type=jnp.float32)
    hn_ref[...] = hn * dis


_BM = 1024
_GRID = (NP // _BM,)

_spec_x = pl.BlockSpec((_BM, D), lambda i: (i, 0))
_spec_w = pl.BlockSpec((D, D), lambda i: (0, 0))
_spec_degp = pl.BlockSpec((NC, _BM), lambda i: (0, i))
_spec_p = pl.BlockSpec((NC, _BM, D), lambda i: (0, i, 0))
_spec_b = pl.BlockSpec((1, D), lambda i: (0, 0))

_mm = pl.pallas_call(
    _mm_body,
    grid=_GRID,
    in_specs=[_spec_x, _spec_w],
    out_specs=_spec_x,
    out_shape=jax.ShapeDtypeStruct((NP, D), jnp.float32),
)

_scale = pl.pallas_call(
    _scale_body,
    grid=_GRID,
    in_specs=[_spec_x, _spec_degp],
    out_specs=_spec_x,
    out_shape=jax.ShapeDtypeStruct((NP, D), jnp.float32),
)

_midf = pl.pallas_call(
    _midf_body,
    grid=_GRID,
    in_specs=[_spec_p, _spec_x, _spec_degp, _spec_b, _spec_w],
    out_specs=[_spec_x, _spec_x],
    out_shape=[jax.ShapeDtypeStruct((NP, D), jnp.float32),
               jax.ShapeDtypeStruct((NP, D), jnp.float32)],
)


def kernel(edge_index, node_feats, edge_feats, nodes_mask_list, W1, b1, W2, b2):
    x = node_feats[-1]
    ei = edge_index[-1]
    ew = edge_feats[-1]
    row, col = ei[0], ei[1]

    ew_bits = lax.bitcast_convert_type(ew, jnp.int32)
    idx = (jnp.stack([row, col, ew_bits])
           .reshape(3, NW, NCH, CH)
           .transpose(1, 2, 0, 3))
    idx = jnp.pad(idx, ((0, 0), (0, 0), (0, 5), (0, 0)))
    x_p = jnp.pad(x, ((0, NP - NN), (0, 0)))

    degp = _deg_kernel(idx)
    h_raw = _mm(x_p, W1)
    h1 = _scale(h_raw, degp)

    Ws = jnp.stack([W2, W2])
    bs = jnp.stack([b1.reshape(1, D), b2.reshape(1, D)])

    def _step(h, wb):
        Wn, b = wb
        p = _agg_kernel(h, idx)
        x2, hn = _midf(p, h, degp, b, Wn)
        return hn, x2

    _, xs = lax.scan(_step, h1, (Ws, bs))
    return xs[-1][:NN]


# speedup vs baseline: 1.0453x; 1.0453x over previous
import dataclasses
import functools

import jax
import jax.numpy as jnp
from jax import lax
from jax.experimental import pallas as pl
from jax.experimental.pallas import tpu as pltpu
from jax.experimental.pallas import tpu_sc as plsc

NN = 10000
NP = 10240
D = 128
EE = 320000
NC = 2
NS = 16
NW = NC * NS
CH = 80
NCH = 125
EPW = NCH * CH
RPT = NP // NS

_mesh = plsc.VectorSubcoreMesh(core_axis_name="c", subcore_axis_name="s")

_cp = pltpu.CompilerParams()
if "needs_layout_passes" in pltpu.CompilerParams.__dataclass_fields__:
    _cp = dataclasses.replace(_cp, needs_layout_passes=False)


@functools.partial(
    pl.kernel,
    out_type=jax.ShapeDtypeStruct((NC, NP), jnp.float32),
    mesh=_mesh,
    compiler_params=_cp,
    scratch_types=[
        pltpu.VMEM((32, CH), jnp.int32),
        pltpu.VMEM((2 * CH,), jnp.float32),
        pltpu.VMEM((RPT,), jnp.float32),
        pltpu.VMEM_SHARED((NP,), jnp.float32),
        pltpu.SemaphoreType.DMA((2,)),
        pltpu.SemaphoreType.DMA((4,)),
    ],
)
def _deg_kernel(idx_hbm, out_hbm, ibuf, stage, zbuf, acc_s, ssem, isem):
    cid = lax.axis_index("c")
    sid = lax.axis_index("s")
    wid = sid * NC + cid

    zeros = jnp.zeros((16,), jnp.float32)

    @pl.loop(0, RPT // 16)
    def _(i):
        zbuf[pl.ds(pl.multiple_of(i * 16, 16), 16)] = zeros

    @pl.loop(0, 2 * CH // 16)
    def _(i):
        stage[pl.ds(pl.multiple_of(i * 16, 16), 16)] = zeros

    base = pl.multiple_of(sid * RPT, RPT)
    pltpu.sync_copy(zbuf, acc_s.at[pl.ds(base, RPT)])
    plsc.subcore_barrier()

    for t in range(4):
        pltpu.async_copy(idx_hbm.at[wid, t], ibuf.at[pl.ds(t * 8, 8)],
                         isem.at[t])
    for b in range(2):
        pltpu.make_async_copy(idx_hbm.at[wid, b],
                              ibuf.at[pl.ds(b * 8, 8)], isem.at[b]).wait()
        pltpu.async_copy(stage.at[pl.ds(b * CH, CH)],
                         acc_s.at[ibuf.at[b * 8 + 1]], ssem.at[b], add=True)

    @pl.loop(0, NCH)
    def _(j):
        b4 = j % 4
        b2 = j % 2
        ir = b4 * 8
        st = stage.at[pl.ds(b2 * CH, CH)]

        @pl.when(j >= 2)
        def _():
            pltpu.make_async_copy(idx_hbm.at[wid, 0],
                                  ibuf.at[pl.ds(b4 * 8, 8)], isem.at[b4]).wait()

        pltpu.make_async_copy(st, acc_s.at[ibuf.at[ir + 1]],
                              ssem.at[b2]).wait()

        @pl.loop(0, CH // 16)
        def _(k):
            sl = pl.ds(pl.multiple_of(k * 16, 16), 16)
            st[sl] = plsc.bitcast(ibuf[ir + 2, sl], jnp.float32)

        pltpu.async_copy(st, acc_s.at[ibuf.at[ir + 1]], ssem.at[b2], add=True)

        @pl.when(j + 4 < NCH)
        def _():
            pltpu.async_copy(idx_hbm.at[wid, j + 4],
                             ibuf.at[pl.ds(b4 * 8, 8)], isem.at[b4])

    for b in range(2):
        pltpu.make_async_copy(stage.at[pl.ds(b * CH, CH)],
                              acc_s.at[ibuf.at[b * 8 + 1]], ssem.at[b]).wait()

    plsc.subcore_barrier()

    pltpu.sync_copy(acc_s.at[pl.ds(base, RPT)], zbuf)
    pltpu.sync_copy(zbuf, out_hbm.at[cid, pl.ds(base, RPT)])


@functools.partial(
    pl.kernel,
    out_type=jax.ShapeDtypeStruct((NC, NP, D), jnp.float32),
    mesh=_mesh,
    compiler_params=_cp,
    scratch_types=[
        pltpu.VMEM((32, CH), jnp.int32),
        pltpu.VMEM((2 * CH, D), jnp.float32),
        pltpu.VMEM((2 * CH, D), jnp.float32),
        pltpu.VMEM_SHARED((NP, D), jnp.float32),
        pltpu.SemaphoreType.DMA((2,)),
        pltpu.SemaphoreType.DMA((2,)),
        pltpu.SemaphoreType.DMA((4,)),
    ],
)
def _agg_kernel(h_hbm, idx_hbm, out_hbm, ibuf, gbuf, sbuf, acc_s,
                gsem, ssem, isem):
    cid = lax.axis_index("c")
    sid = lax.axis_index("s")
    wid = sid * NC + cid

    zeros = jnp.zeros((16,), jnp.float32)

    for t in range(4):
        pltpu.async_copy(idx_hbm.at[wid, t], ibuf.at[pl.ds(t * 8, 8)],
                         isem.at[t])

    @pl.loop(0, 2 * CH)
    def _(r):
        for f in range(D // 16):
            sbuf[r, pl.ds(f * 16, 16)] = zeros

    @pl.loop(0, RPT // CH)
    def _(t):
        base = pl.multiple_of(sid * RPT + t * CH, CH)
        pltpu.async_copy(sbuf.at[pl.ds(0, CH)], acc_s.at[pl.ds(base, CH)],
                         gsem.at[0])

    @pl.loop(0, RPT // CH)
    def _(t):
        base = pl.multiple_of(sid * RPT + t * CH, CH)
        pltpu.make_async_copy(sbuf.at[pl.ds(0, CH)],
                              acc_s.at[pl.ds(base, CH)], gsem.at[0]).wait()

    plsc.subcore_barrier()

    for b in range(2):
        pltpu.make_async_copy(idx_hbm.at[wid, b],
                              ibuf.at[pl.ds(b * 8, 8)], isem.at[b]).wait()
        pltpu.async_copy(h_hbm.at[ibuf.at[b * 8]],
                         gbuf.at[pl.ds(b * CH, CH)], gsem.at[b])
        pltpu.async_copy(sbuf.at[pl.ds(b * CH, CH)],
                         acc_s.at[ibuf.at[b * 8 + 1]], ssem.at[b], add=True)

    @pl.loop(0, NCH)
    def _(j):
        b4 = j % 4
        b2 = j % 2
        g = gbuf.at[pl.ds(b2 * CH, CH)]
        s = sbuf.at[pl.ds(b2 * CH, CH)]
        ir = b4 * 8
        pltpu.make_async_copy(h_hbm.at[ibuf.at[ir]], g, gsem.at[b2]).wait()
        pltpu.make_async_copy(s, acc_s.at[ibuf.at[ir + 1]],
                              ssem.at[b2]).wait()

        @plsc.parallel_loop(0, CH // 16, unroll=5)
        def _(k):
            sl16 = pl.ds(pl.multiple_of(k * 16, 16), 16)
            wv = plsc.bitcast(ibuf[ir + 2, sl16], jnp.float32)
            for l in range(16):
                w = wv[l]
                e = k * 16 + l
                for f in range(D // 16):
                    sl = pl.ds(f * 16, 16)
                    s[e, sl] = g[e, sl] * w

        @pl.when(j + 2 < NCH)
        def _():
            bn = (b4 + 2) % 4
            pltpu.make_async_copy(idx_hbm.at[wid, 0],
                                  ibuf.at[pl.ds(bn * 8, 8)], isem.at[bn]).wait()
            pltpu.async_copy(h_hbm.at[ibuf.at[bn * 8]], g, gsem.at[b2])

        pltpu.async_copy(s, acc_s.at[ibuf.at[ir + 1]], ssem.at[b2], add=True)

        @pl.when(j + 4 < NCH)
        def _():
            pltpu.async_copy(idx_hbm.at[wid, j + 4],
                             ibuf.at[pl.ds(b4 * 8, 8)], isem.at[b4])

    for b in range(2):
        pltpu.make_async_copy(sbuf.at[pl.ds(b * CH, CH)],
                              acc_s.at[ibuf.at[b * 8 + 1]], ssem.at[b]).wait()

    plsc.subcore_barrier()

    base = pl.multiple_of(sid * RPT, RPT)
    pltpu.sync_copy(acc_s.at[pl.ds(base, RPT)],
                    out_hbm.at[cid, pl.ds(base, RPT)])


def _pre_body(x_ref, w_ref, degp_ref, h_ref):
    deg = 1.0 + jnp.sum(degp_ref[...], axis=0)
    dis = lax.rsqrt(deg)
    h = jnp.dot(x_ref[...], w_ref[...], preferred_element_type=jnp.float32)
    h_ref[...] = h * dis[:, None]


def _post_body(p_ref, h_ref, degp_ref, b_ref, o_ref):
    deg = 1.0 + jnp.sum(degp_ref[...], axis=0)
    dis = lax.rsqrt(deg)[:, None]
    o_ref[...] = jnp.maximum(dis * (p_ref[0] + p_ref[1] + h_ref[...]) + b_ref[...], 0.0)


_BM = 1024
_GRID = (NP // _BM,)

_spec_x = pl.BlockSpec((_BM, D), lambda i: (i, 0))
_spec_w = pl.BlockSpec((D, D), lambda i: (0, 0))
_spec_degp = pl.BlockSpec((NC, _BM), lambda i: (0, i))
_spec_p = pl.BlockSpec((NC, _BM, D), lambda i: (0, i, 0))
_spec_b = pl.BlockSpec((1, D), lambda i: (0, 0))

_pre = pl.pallas_call(
    _pre_body,
    grid=_GRID,
    in_specs=[_spec_x, _spec_w, _spec_degp],
    out_specs=_spec_x,
    out_shape=jax.ShapeDtypeStruct((NP, D), jnp.float32),
)

_post = pl.pallas_call(
    _post_body,
    grid=_GRID,
    in_specs=[_spec_p, _spec_x, _spec_degp, _spec_b],
    out_specs=_spec_x,
    out_shape=jax.ShapeDtypeStruct((NP, D), jnp.float32),
)


def kernel(edge_index, node_feats, edge_feats, nodes_mask_list, W1, b1, W2, b2):
    x = node_feats[-1]
    ei = edge_index[-1]
    ew = edge_feats[-1]
    row, col = ei[0], ei[1]

    ew_bits = lax.bitcast_convert_type(ew, jnp.int32)
    idx = (jnp.stack([row, col, ew_bits])
           .reshape(3, NW, NCH, CH)
           .transpose(1, 2, 0, 3))
    idx = jnp.pad(idx, ((0, 0), (0, 0), (0, 5), (0, 0)))
    x_p = jnp.pad(x, ((0, NP - NN), (0, 0)))

    degp = _deg_kernel(idx)
    Ws = jnp.stack([W1, W2])
    bs = jnp.stack([b1.reshape(1, D), b2.reshape(1, D)])

    def _step(xc, wb):
        W, b = wb
        h = _pre(xc, W, degp)
        p = _agg_kernel(h, idx)
        return _post(p, h, degp, b), None

    xf, _ = lax.scan(_step, x_p, (Ws, bs))
    return xf[:NN]


# speedup vs baseline: 1.0534x; 1.0078x over previous
import dataclasses
import functools

import jax
import jax.numpy as jnp
from jax import lax
from jax.experimental import pallas as pl
from jax.experimental.pallas import tpu as pltpu
from jax.experimental.pallas import tpu_sc as plsc

NN = 10000
NP = 10240
D = 128
EE = 320000
NC = 2
NS = 16
NW = NC * NS
CH = 80
NCH = 125
EPW = NCH * CH
RPT = NP // NS

_mesh = plsc.VectorSubcoreMesh(core_axis_name="c", subcore_axis_name="s")

_cp = pltpu.CompilerParams()
if "needs_layout_passes" in pltpu.CompilerParams.__dataclass_fields__:
    _cp = dataclasses.replace(_cp, needs_layout_passes=False)


@functools.partial(
    pl.kernel,
    out_type=jax.ShapeDtypeStruct((NC, NP), jnp.float32),
    mesh=_mesh,
    compiler_params=_cp,
    scratch_types=[
        pltpu.VMEM((32, CH), jnp.int32),
        pltpu.VMEM((2 * CH,), jnp.float32),
        pltpu.VMEM((RPT,), jnp.float32),
        pltpu.VMEM_SHARED((NP,), jnp.float32),
        pltpu.SemaphoreType.DMA((2,)),
        pltpu.SemaphoreType.DMA((4,)),
    ],
)
def _deg_kernel(idx_hbm, out_hbm, ibuf, stage, zbuf, acc_s, ssem, isem):
    cid = lax.axis_index("c")
    sid = lax.axis_index("s")
    wid = sid * NC + cid

    zeros = jnp.zeros((16,), jnp.float32)

    @pl.loop(0, RPT // 16)
    def _(i):
        zbuf[pl.ds(pl.multiple_of(i * 16, 16), 16)] = zeros

    @pl.loop(0, 2 * CH // 16)
    def _(i):
        stage[pl.ds(pl.multiple_of(i * 16, 16), 16)] = zeros

    base = pl.multiple_of(sid * RPT, RPT)
    pltpu.sync_copy(zbuf, acc_s.at[pl.ds(base, RPT)])
    plsc.subcore_barrier()

    for t in range(4):
        pltpu.async_copy(idx_hbm.at[wid, t], ibuf.at[pl.ds(t * 8, 8)],
                         isem.at[t])
    for b in range(2):
        pltpu.make_async_copy(idx_hbm.at[wid, b],
                              ibuf.at[pl.ds(b * 8, 8)], isem.at[b]).wait()
        pltpu.async_copy(stage.at[pl.ds(b * CH, CH)],
                         acc_s.at[ibuf.at[b * 8 + 1]], ssem.at[b], add=True)

    @pl.loop(0, NCH)
    def _(j):
        b4 = j % 4
        b2 = j % 2
        ir = b4 * 8
        st = stage.at[pl.ds(b2 * CH, CH)]

        @pl.when(j >= 2)
        def _():
            pltpu.make_async_copy(idx_hbm.at[wid, 0],
                                  ibuf.at[pl.ds(b4 * 8, 8)], isem.at[b4]).wait()

        pltpu.make_async_copy(st, acc_s.at[ibuf.at[ir + 1]],
                              ssem.at[b2]).wait()

        @pl.loop(0, CH // 16)
        def _(k):
            sl = pl.ds(pl.multiple_of(k * 16, 16), 16)
            st[sl] = plsc.bitcast(ibuf[ir + 2, sl], jnp.float32)

        pltpu.async_copy(st, acc_s.at[ibuf.at[ir + 1]], ssem.at[b2], add=True)

        @pl.when(j + 4 < NCH)
        def _():
            pltpu.async_copy(idx_hbm.at[wid, j + 4],
                             ibuf.at[pl.ds(b4 * 8, 8)], isem.at[b4])

    for b in range(2):
        pltpu.make_async_copy(stage.at[pl.ds(b * CH, CH)],
                              acc_s.at[ibuf.at[b * 8 + 1]], ssem.at[b]).wait()

    plsc.subcore_barrier()

    pltpu.sync_copy(acc_s.at[pl.ds(base, RPT)], zbuf)
    pltpu.sync_copy(zbuf, out_hbm.at[cid, pl.ds(base, RPT)])


@functools.partial(
    pl.kernel,
    out_type=jax.ShapeDtypeStruct((NC, NP, D), jnp.float32),
    mesh=_mesh,
    compiler_params=_cp,
    scratch_types=[
        pltpu.VMEM((48, CH), jnp.int32),
        pltpu.VMEM((2 * CH, D), jnp.float32),
        pltpu.VMEM((2 * CH, D), jnp.float32),
        pltpu.VMEM_SHARED((NP, D), jnp.float32),
        pltpu.SemaphoreType.DMA((2,)),
        pltpu.SemaphoreType.DMA((2,)),
        pltpu.SemaphoreType.DMA((6,)),
    ],
)
def _agg_kernel(h_hbm, idx_hbm, out_hbm, ibuf, gbuf, sbuf, acc_s,
                gsem, ssem, isem):
    cid = lax.axis_index("c")
    sid = lax.axis_index("s")
    wid = sid * NC + cid

    zeros = jnp.zeros((16,), jnp.float32)

    for t in range(4):
        pltpu.async_copy(idx_hbm.at[wid, t], ibuf.at[pl.ds(t * 8, 8)],
                         isem.at[t])

    @pl.loop(0, 2 * CH)
    def _(r):
        for f in range(D // 16):
            sbuf[r, pl.ds(f * 16, 16)] = zeros

    @pl.loop(0, RPT // CH)
    def _(t):
        base = pl.multiple_of(sid * RPT + t * CH, CH)
        pltpu.async_copy(sbuf.at[pl.ds(0, CH)], acc_s.at[pl.ds(base, CH)],
                         gsem.at[0])

    @pl.loop(0, RPT // CH)
    def _(t):
        base = pl.multiple_of(sid * RPT + t * CH, CH)
        pltpu.make_async_copy(sbuf.at[pl.ds(0, CH)],
                              acc_s.at[pl.ds(base, CH)], gsem.at[0]).wait()

    plsc.subcore_barrier()

    for b in range(2):
        pltpu.make_async_copy(idx_hbm.at[wid, b],
                              ibuf.at[pl.ds(b * 8, 8)], isem.at[b]).wait()
        pltpu.async_copy(h_hbm.at[ibuf.at[b * 8]],
                         gbuf.at[pl.ds(b * CH, CH)], gsem.at[b])
        pltpu.async_copy(sbuf.at[pl.ds(b * CH, CH)],
                         acc_s.at[ibuf.at[b * 8 + 1]], ssem.at[b], add=True)

    @pl.loop(0, NCH)
    def _(j):
        b6 = j % 6
        b2 = j % 2
        g = gbuf.at[pl.ds(b2 * CH, CH)]
        s = sbuf.at[pl.ds(b2 * CH, CH)]
        ir = b6 * 8
        pltpu.make_async_copy(h_hbm.at[ibuf.at[ir]], g, gsem.at[b2]).wait()
        pltpu.make_async_copy(s, acc_s.at[ibuf.at[ir + 1]],
                              ssem.at[b2]).wait()

        @pl.when(j + 4 < NCH)
        def _():
            b_f = (b6 + 4) % 6
            pltpu.async_copy(idx_hbm.at[wid, j + 4],
                             ibuf.at[pl.ds(b_f * 8, 8)], isem.at[b_f])

        @plsc.parallel_loop(0, CH // 16, unroll=5)
        def _(k):
            sl16 = pl.ds(pl.multiple_of(k * 16, 16), 16)
            wv = plsc.bitcast(ibuf[ir + 2, sl16], jnp.float32)
            for l in range(16):
                w = wv[l]
                e = k * 16 + l
                for f in range(D // 16):
                    sl = pl.ds(f * 16, 16)
                    s[e, sl] = g[e, sl] * w

        @pl.when(j + 2 < NCH)
        def _():
            bn = (b6 + 2) % 6
            pltpu.make_async_copy(idx_hbm.at[wid, 0],
                                  ibuf.at[pl.ds(bn * 8, 8)], isem.at[bn]).wait()
            pltpu.async_copy(h_hbm.at[ibuf.at[bn * 8]], g, gsem.at[b2])

        pltpu.async_copy(s, acc_s.at[ibuf.at[ir + 1]], ssem.at[b2], add=True)

    for b in range(2):
        pltpu.make_async_copy(sbuf.at[pl.ds(b * CH, CH)],
                              acc_s.at[ibuf.at[b * 8 + 1]], ssem.at[b]).wait()

    plsc.subcore_barrier()

    base = pl.multiple_of(sid * RPT, RPT)
    pltpu.sync_copy(acc_s.at[pl.ds(base, RPT)],
                    out_hbm.at[cid, pl.ds(base, RPT)])


def _pre_body(x_ref, w_ref, degp_ref, h_ref):
    deg = 1.0 + jnp.sum(degp_ref[...], axis=0)
    dis = lax.rsqrt(deg)
    h = jnp.dot(x_ref[...], w_ref[...], preferred_element_type=jnp.float32)
    h_ref[...] = h * dis[:, None]


def _post_body(p_ref, h_ref, degp_ref, b_ref, o_ref):
    deg = 1.0 + jnp.sum(degp_ref[...], axis=0)
    dis = lax.rsqrt(deg)[:, None]
    o_ref[...] = jnp.maximum(dis * (p_ref[0] + p_ref[1] + h_ref[...]) + b_ref[...], 0.0)


_BM = 1024
_GRID = (NP // _BM,)

_spec_x = pl.BlockSpec((_BM, D), lambda i: (i, 0))
_spec_w = pl.BlockSpec((D, D), lambda i: (0, 0))
_spec_degp = pl.BlockSpec((NC, _BM), lambda i: (0, i))
_spec_p = pl.BlockSpec((NC, _BM, D), lambda i: (0, i, 0))
_spec_b = pl.BlockSpec((1, D), lambda i: (0, 0))

_pre = pl.pallas_call(
    _pre_body,
    grid=_GRID,
    in_specs=[_spec_x, _spec_w, _spec_degp],
    out_specs=_spec_x,
    out_shape=jax.ShapeDtypeStruct((NP, D), jnp.float32),
)

_post = pl.pallas_call(
    _post_body,
    grid=_GRID,
    in_specs=[_spec_p, _spec_x, _spec_degp, _spec_b],
    out_specs=_spec_x,
    out_shape=jax.ShapeDtypeStruct((NP, D), jnp.float32),
)


def kernel(edge_index, node_feats, edge_feats, nodes_mask_list, W1, b1, W2, b2):
    x = node_feats[-1]
    ei = edge_index[-1]
    ew = edge_feats[-1]
    row, col = ei[0], ei[1]

    ew_bits = lax.bitcast_convert_type(ew, jnp.int32)
    idx = (jnp.stack([row, col, ew_bits])
           .reshape(3, NW, NCH, CH)
           .transpose(1, 2, 0, 3))
    idx = jnp.pad(idx, ((0, 0), (0, 0), (0, 5), (0, 0)))
    x_p = jnp.pad(x, ((0, NP - NN), (0, 0)))

    degp = _deg_kernel(idx)
    Ws = jnp.stack([W1, W2])
    bs = jnp.stack([b1.reshape(1, D), b2.reshape(1, D)])

    def _step(xc, wb):
        W, b = wb
        h = _pre(xc, W, degp)
        p = _agg_kernel(h, idx)
        return _post(p, h, degp, b), None

    xf, _ = lax.scan(_step, x_p, (Ws, bs))
    return xf[:NN]
